# baseline (speedup 1.0000x reference)
import jax
import jax.numpy as jnp
from jax import lax
from jax.experimental import pallas as pl
from jax.experimental.pallas import tpu as pltpu
from jax.experimental.pallas import tpu_sc as plsc

N = 10000
E = 160000
MAXN = 9
D = 64
NB = 4
NELEM = 89
CUT = 5.0

NC = 2
NS = 16
NW = NC * NS
K = 128
CPC = 8
E_PAD = 163840
EPW = E_PAD // NW
NCALL = EPW // (CPC * K)
N_PAD = 10240
NPT = N_PAD // NS
GAP0 = 5120
GAPADD = 16
N_ALLOC = N_PAD + GAPADD


def _embed_body(types_ref, emb_ref, x_ref):
    t = types_ref[...]
    iota = lax.broadcasted_iota(jnp.int32, (N, NELEM), 1)
    oh = (iota == t).astype(jnp.float32)
    x_ref[...] = jnp.dot(oh, emb_ref[...], preferred_element_type=jnp.float32)


def _embed(atom_types, atom_emb):
    return pl.pallas_call(
        _embed_body,
        out_shape=jax.ShapeDtypeStruct((N, D), jnp.float32),
    )(atom_types.reshape(N, 1).astype(jnp.int32), atom_emb)


_EB = 2048


def _prep_body(bd_ref, wbe_ref, wbw_ref, wc_ref, wg_ref,
               b0_ref, b1_ref, b2_ref, b3_ref, bw_ref):
    pid = pl.program_id(0)
    r = bd_ref[0]
    scale = jnp.sqrt(2.0 / CUT) / r
    rows = [jnp.sin((float(k + 1) * jnp.pi / CUT) * r) * scale for k in range(MAXN)]
    rbf = jnp.concatenate(rows, axis=0)
    wbe = wbe_ref[...]
    cols = []
    for i in range(NB):
        cols.append(jnp.dot(wbe, wc_ref[i, 2 * D:, :], preferred_element_type=jnp.float32))
        cols.append(jnp.dot(wbe, wg_ref[i, 2 * D:, :], preferred_element_type=jnp.float32))
    w2 = jnp.concatenate(cols, axis=1)
    big = lax.dot_general(rbf, w2, (((0,), (0,)), ((), ())),
                          preferred_element_type=jnp.float32)
    outs = (b0_ref, b1_ref, b2_ref, b3_ref)
    for i in range(NB):
        outs[i][...] = big[:, i * 2 * D:(i + 1) * 2 * D]
    bw = lax.dot_general(rbf, wbw_ref[...], (((0,), (0,)), ((), ())),
                         preferred_element_type=jnp.float32)
    erow = pid * _EB + lax.broadcasted_iota(jnp.int32, (_EB, D), 0)
    bw = jnp.where(erow < E, bw, 0.0)
    bw_ref[...] = jnp.concatenate([bw, jnp.zeros((_EB, D), jnp.float32)], axis=1)


def _prep(bond_dist, W_be, W_bw, Wc, Wg):
    bd = jnp.pad(bond_dist, (0, E_PAD - E), constant_values=1.0)
    bd = bd.reshape(E_PAD // _EB, 1, _EB)
    full = lambda s: pl.BlockSpec(s, lambda e: (0,) * len(s))
    out_spec = pl.BlockSpec((_EB, 2 * D), lambda e: (e, 0))
    return pl.pallas_call(
        _prep_body,
        grid=(E_PAD // _EB,),
        in_specs=[pl.BlockSpec((1, 1, _EB), lambda e: (e, 0, 0)),
                  full((MAXN, D)), full((MAXN, D)),
                  full((NB, 3 * D, D)), full((NB, 3 * D, D))],
        out_specs=[out_spec] * (NB + 1),
        out_shape=[jax.ShapeDtypeStruct((E_PAD, 2 * D), jnp.float32)] * (NB + 1),
    )(bd, W_be, W_bw, Wc, Wg)


def _proj_body(x_ref, agg_ref, wc_ref, wg_ref, bbe_ref, bc_ref, bg_ref,
               xn_ref, ts_ref, td_ref):
    x = x_ref[...]
    if agg_ref is not None:
        x = x + agg_ref[:N]
    xn_ref[...] = x
    wc = wc_ref[...]
    wg = wg_ref[...]
    wsrc = jnp.concatenate([wc[:D], wg[:D]], axis=1)
    wdst = jnp.concatenate([wc[D:2 * D], wg[D:2 * D]], axis=1)
    bbe = bbe_ref[...]
    cb = jnp.dot(bbe, wc[2 * D:], preferred_element_type=jnp.float32) + bc_ref[...]
    gb = jnp.dot(bbe, wg[2 * D:], preferred_element_type=jnp.float32) + bg_ref[...]
    ts_ref[...] = jnp.dot(x, wsrc, preferred_element_type=jnp.float32)
    td_ref[...] = jnp.dot(x, wdst, preferred_element_type=jnp.float32) \
        + jnp.concatenate([cb, gb], axis=1)


def _proj(x, agg, Wc_i, Wg_i, b_be64, bc_i, bg_i):
    args = (x,) + ((agg,) if agg is not None else ()) + (
        Wc_i, Wg_i, b_be64.reshape(1, D), bc_i.reshape(1, D), bg_i.reshape(1, D))
    if agg is not None:
        def body(x_ref, agg_ref, wc_ref, wg_ref, bbe_ref, bc_ref, bg_ref,
                 xn_ref, ts_ref, td_ref):
            return _proj_body(x_ref, agg_ref, wc_ref, wg_ref, bbe_ref, bc_ref,
                              bg_ref, xn_ref, ts_ref, td_ref)
    else:
        def body(x_ref, wc_ref, wg_ref, bbe_ref, bc_ref, bg_ref,
                 xn_ref, ts_ref, td_ref):
            return _proj_body(x_ref, None, wc_ref, wg_ref, bbe_ref, bc_ref,
                              bg_ref, xn_ref, ts_ref, td_ref)
    return pl.pallas_call(
        body,
        out_shape=[jax.ShapeDtypeStruct((N, D), jnp.float32),
                   jax.ShapeDtypeStruct((N, 2 * D), jnp.float32),
                   jax.ShapeDtypeStruct((N, 2 * D), jnp.float32)],
    )(*args)


EPC = NW * CPC * K


def _sc_body(call_idx, ts_hbm, td_hbm, bb_hbm, bw_hbm, ei_hbm,
             out_hbm, sidx, didx, Sv, Dv, efv, bwv, msgv,
             sem_s, sem_d, sem_b, sem_w):
    c = lax.axis_index("c")
    s = lax.axis_index("s")
    wid = s * NC + c

    for ci in range(CPC):
        local = wid * (CPC * K) + ci * K
        base = call_idx * EPC + local
        pltpu.sync_copy(ei_hbm.at[0, pl.ds(base, K)], sidx)
        pltpu.sync_copy(ei_hbm.at[1, pl.ds(base, K)], didx)
        pltpu.async_copy(ts_hbm.at[sidx], Sv, sem_s).wait()
        pltpu.async_copy(td_hbm.at[didx], Dv, sem_d).wait()
        pltpu.async_copy(bb_hbm.at[pl.ds(base, K)], efv, sem_b).wait()
        pltpu.async_copy(bw_hbm.at[pl.ds(base, K)], bwv, sem_w).wait()

        def edge(e, _):
            for q in range(D // 16):
                dsc = pl.ds(q * 16, 16)
                dsg = pl.ds(D + q * 16, 16)
                c_ = Sv[e, dsc] + Dv[e, dsc] + efv[e, dsc]
                g_ = Sv[e, dsg] + Dv[e, dsg] + efv[e, dsg]
                den = (1.0 + jnp.exp(-c_)) * (1.0 + jnp.exp(-g_))
                msgv[e, dsc] = c_ * bwv[e, dsc] / den
            return 0

        lax.fori_loop(0, K, edge, 0)
        pltpu.sync_copy(msgv, out_hbm.at[pl.ds(local, K)])


def _sc_call(call_idx, tsrc, tdst, bb_i, bwp, ei_pad):
    mesh = plsc.VectorSubcoreMesh(core_axis_name="c", subcore_axis_name="s")

    def body(ts_hbm, td_hbm, bb_hbm, bw_hbm, ei_hbm, out_hbm, *rest):
        return _sc_body(call_idx, ts_hbm, td_hbm, bb_hbm, bw_hbm,
                        ei_hbm, out_hbm, *rest)

    fn = pl.kernel(
        body,
        out_type=jax.ShapeDtypeStruct((EPC, D), jnp.float32),
        mesh=mesh,
        scratch_types=[
            pltpu.VMEM((K,), jnp.int32),
            pltpu.VMEM((K,), jnp.int32),
            pltpu.VMEM((K, 2 * D), jnp.float32),
            pltpu.VMEM((K, 2 * D), jnp.float32),
            pltpu.VMEM((K, 2 * D), jnp.float32),
            pltpu.VMEM((K, 2 * D), jnp.float32),
            pltpu.VMEM((K, D), jnp.float32),
            pltpu.SemaphoreType.DMA,
            pltpu.SemaphoreType.DMA,
            pltpu.SemaphoreType.DMA,
            pltpu.SemaphoreType.DMA,
        ],
    )
    return fn(tsrc, tdst, bb_i, bwp, ei_pad)


_GPC = EPC // K


def _segsum_body(di_ref, msg_ref, out_ref):
    @pl.when(pl.program_id(0) == 0)
    def _():
        def zbody(r, _):
            out_ref[r] = jnp.zeros((8, D), jnp.float32)
            return 0

        lax.fori_loop(0, N_PAD // 8, zbody, 0)

    iota8 = lax.broadcasted_iota(jnp.int32, (8, 1), 0)

    def body(g, _):
        dvec = di_ref[g]
        mblk = msg_ref[pl.ds(pl.multiple_of(g * K, 8), K), :]
        for l in range(K):
            d = dvec[0, l]
            hi = lax.div(d, 8)
            lo = lax.rem(d, 8)
            upd = jnp.where(iota8 == lo, mblk[l][None, :], 0.0)
            out_ref[hi] += upd
        return 0

    lax.fori_loop(0, _GPC, body, 0)


def _segsum(msgs, ei_pad):
    di = ei_pad[1].reshape(E_PAD // K, 1, K)
    msg = jnp.concatenate(msgs, axis=0)
    out = pl.pallas_call(
        _segsum_body,
        grid=(NCALL,),
        in_specs=[pl.BlockSpec((_GPC, 1, K), lambda p: (p, 0, 0)),
                  pl.BlockSpec((EPC, D), lambda p: (p, 0))],
        out_specs=pl.BlockSpec((N_PAD // 8, 8, D), lambda p: (0, 0, 0)),
        out_shape=jax.ShapeDtypeStruct((N_PAD // 8, 8, D), jnp.float32),
        compiler_params=pltpu.CompilerParams(
            dimension_semantics=("arbitrary",)),
    )(di, msg)
    return out.reshape(N_PAD, D)


def _sc_block(tsrc, tdst, bb_i, bwp, ei_pad):
    msgs = [_sc_call(j, tsrc, tdst, bb_i, bwp, ei_pad) for j in range(NCALL)]
    return _segsum(msgs, ei_pad)


def _readout_body(x_ref, agg_ref, w1_ref, b1_ref, w2_ref, b2_ref, w3_ref, b3_ref,
                  out_ref):
    x = x_ref[...] + agg_ref[:N]
    h = jnp.dot(x, w1_ref[...], preferred_element_type=jnp.float32) + b1_ref[...]
    h = h * jax.nn.sigmoid(h)
    h = jnp.dot(h, w2_ref[...], preferred_element_type=jnp.float32) + b2_ref[...]
    h = h * jax.nn.sigmoid(h)
    srow = jnp.sum(h, axis=0, keepdims=True)
    out_ref[...] = jnp.dot(srow, w3_ref[...], preferred_element_type=jnp.float32) \
        + float(N) * b3_ref[...]


def _readout(x, agg, Wro1, bro1, Wro2, bro2, Wro3, bro3):
    return pl.pallas_call(
        _readout_body,
        out_shape=jax.ShapeDtypeStruct((1, 1), jnp.float32),
    )(x, agg, Wro1, bro1.reshape(1, D), Wro2, bro2.reshape(1, D),
      Wro3, bro3.reshape(1, 1))


def kernel(atom_types, edge_index, bond_dist, atom_emb, freqs, W_be, b_be, W_bw,
           Wc, bc, Wg, bg, Wro1, bro1, Wro2, bro2, Wro3, bro3):
    ei = jnp.pad(edge_index.astype(jnp.int32), ((0, 0), (0, E_PAD - E)))
    preps = _prep(bond_dist, W_be, W_bw, Wc, Wg)
    bb = preps[:NB]
    bwp = preps[NB]
    x = _embed(atom_types, atom_emb)
    agg = None
    for i in range(NB):
        x, tsrc, tdst = _proj(x, agg, Wc[i], Wg[i], b_be, bc[i], bg[i])
        agg = _sc_block(tsrc, tdst, bb[i], bwp, ei)
    out = _readout(x, agg, Wro1, bro1, Wro2, bro2, Wro3, bro3)
    return out.reshape(1)
